# baseline (speedup 1.0000x reference)
import jax
import jax.numpy as jnp
from jax import lax
from jax.experimental import pallas as pl
from jax.experimental.pallas import tpu as pltpu
from jax.experimental.pallas import tpu_sc as plsc

B = 4096
L = 200
D = 64
VOCAB = 1000000
NC = 2
NS = 16
NW = NC * NS
BPW = B // NW
HALF = L // 2


def _disc_body(x_hbm, table_hbm, w_hbm, bias_hbm, out_hbm,
               idx_v, rows_v, w_v, bias_v, out_v, sem):
    c = lax.axis_index("c")
    s = lax.axis_index("s")
    wid = s * NC + c
    base = wid * BPW

    pltpu.sync_copy(x_hbm.at[pl.ds(base, BPW)], idx_v)
    pltpu.sync_copy(w_hbm, w_v)
    pltpu.sync_copy(bias_hbm, bias_v)

    w0 = w_v[pl.ds(0, 16)]
    w1 = w_v[pl.ds(16, 16)]
    w2 = w_v[pl.ds(32, 16)]
    w3 = w_v[pl.ds(48, 16)]
    bv = bias_v[...]
    lane = lax.iota(jnp.int32, 16)
    lane0 = lane == 0
    zero = jnp.zeros((16,), jnp.float32)
    inv_l = jnp.float32(1.0 / L)

    dnums = lax.GatherDimensionNumbers(
        offset_dims=(), collapsed_slice_dims=(0,), start_index_map=(0,))

    def group_body(g, carry):
        def row_body(j, zgroup):
            b = g * 16 + j
            cp0 = pltpu.async_copy(table_hbm.at[idx_v.at[b, 0]],
                                   rows_v.at[pl.ds(0, HALF)], sem)
            cp1 = pltpu.async_copy(table_hbm.at[idx_v.at[b, 1]],
                                   rows_v.at[pl.ds(HALF, HALF)], sem)
            cp0.wait()
            cp1.wait()

            def acc_body(l, acc):
                a0, a1, a2, a3 = acc
                return (a0 + rows_v[l, pl.ds(0, 16)],
                        a1 + rows_v[l, pl.ds(16, 16)],
                        a2 + rows_v[l, pl.ds(32, 16)],
                        a3 + rows_v[l, pl.ds(48, 16)])

            a0, a1, a2, a3 = lax.fori_loop(0, L, acc_body,
                                           (zero, zero, zero, zero))
            zv = (a0 * w0 + a1 * w1 + a2 * w2 + a3 * w3) * inv_l + bv
            for k in (1, 2, 4, 8):
                shuf = lax.gather(zv, (lane ^ k)[:, None], dnums,
                                  slice_sizes=(1,),
                                  mode=lax.GatherScatterMode.PROMISE_IN_BOUNDS)
                zv = zv + shuf
            return jnp.where(lane == j, zv, zgroup)

        zgroup = lax.fori_loop(0, 16, row_body, zero)
        out_v[pl.ds(g * 16, 16)] = 1.0 / (1.0 + jnp.exp(-zgroup))
        return carry

    lax.fori_loop(0, BPW // 16, group_body, 0)
    pltpu.sync_copy(out_v, out_hbm.at[pl.ds(base, BPW)])


def kernel(x, embed_table, fc_w, fc_b):
    x3 = x.astype(jnp.int32).reshape(B, 2, HALF)
    w = fc_w.reshape(D).astype(jnp.float32)
    bpad = jnp.pad(fc_b.astype(jnp.float32), (0, 15))
    mesh = plsc.VectorSubcoreMesh(core_axis_name="c", subcore_axis_name="s")
    run = pl.kernel(
        _disc_body,
        out_type=jax.ShapeDtypeStruct((B,), jnp.float32),
        mesh=mesh,
        compiler_params=pltpu.CompilerParams(use_tc_tiling_on_sc=False),
        scratch_types=[
            pltpu.VMEM((BPW, 2, HALF), jnp.int32),
            pltpu.VMEM((L, D), jnp.float32),
            pltpu.VMEM((D,), jnp.float32),
            pltpu.VMEM((16,), jnp.float32),
            pltpu.VMEM((BPW,), jnp.float32),
            pltpu.SemaphoreType.DMA,
        ],
    )
    out = run(x3, embed_table, w, bpad)
    return out.reshape(B, 1)


# speedup vs baseline: 1.2419x; 1.2419x over previous
import jax
import jax.numpy as jnp
from jax import lax
from jax.experimental import pallas as pl
from jax.experimental.pallas import tpu as pltpu
from jax.experimental.pallas import tpu_sc as plsc

B = 4096
L = 200
D = 64
VOCAB = 1000000
NC = 2
NS = 16
NW = NC * NS
BPW = B // NW
HALF = L // 2


NBUF = 4


def _disc_body(x_hbm, table_hbm, w_hbm, bias_hbm, out_hbm,
               idx_v, rows0, rows1, rows2, rows3, w_v, bias_v, out_v,
               sem0, sem1, sem2, sem3):
    c = lax.axis_index("c")
    s = lax.axis_index("s")
    wid = s * NC + c
    base = wid * BPW

    bufs = ((rows0, sem0), (rows1, sem1), (rows2, sem2), (rows3, sem3))

    pltpu.sync_copy(x_hbm.at[pl.ds(base, BPW)], idx_v)
    pltpu.sync_copy(w_hbm, w_v)
    pltpu.sync_copy(bias_hbm, bias_v)

    w0 = w_v[pl.ds(0, 16)]
    w1 = w_v[pl.ds(16, 16)]
    w2 = w_v[pl.ds(32, 16)]
    w3 = w_v[pl.ds(48, 16)]
    bv = bias_v[...]
    lane = lax.iota(jnp.int32, 16)
    lane0 = lane == 0
    zero = jnp.zeros((16,), jnp.float32)
    inv_l = jnp.float32(1.0 / L)

    dnums = lax.GatherDimensionNumbers(
        offset_dims=(), collapsed_slice_dims=(0,), start_index_map=(0,))

    def fire(b, buf, sem):
        pltpu.async_copy(table_hbm.at[idx_v.at[b, 0]],
                         buf.at[pl.ds(0, HALF)], sem)
        pltpu.async_copy(table_hbm.at[idx_v.at[b, 1]],
                         buf.at[pl.ds(HALF, HALF)], sem)

    def drain(buf, sem):
        pltpu.make_async_copy(table_hbm.at[pl.ds(0, L)], buf, sem).wait()

    for p in range(NBUF - 1):
        fire(p, *bufs[p])

    def group_body(g, carry):
        zgroup = zero
        for p in range(16):
            b = g * 16 + p
            nxt = b + NBUF - 1
            buf, sem = bufs[p % NBUF]

            @pl.when(nxt < BPW)
            def _():
                fire(nxt, *bufs[(p + NBUF - 1) % NBUF])

            drain(buf, sem)

            def acc_body(l2, acc):
                a0, a1, a2, a3, a4, a5, a6, a7 = acc
                i = l2 * 2
                return (a0 + buf[i, pl.ds(0, 16)],
                        a1 + buf[i, pl.ds(16, 16)],
                        a2 + buf[i, pl.ds(32, 16)],
                        a3 + buf[i, pl.ds(48, 16)],
                        a4 + buf[i + 1, pl.ds(0, 16)],
                        a5 + buf[i + 1, pl.ds(16, 16)],
                        a6 + buf[i + 1, pl.ds(32, 16)],
                        a7 + buf[i + 1, pl.ds(48, 16)])

            a0, a1, a2, a3, a4, a5, a6, a7 = lax.fori_loop(
                0, L // 2, acc_body, (zero,) * 8)
            zv = ((a0 + a4) * w0 + (a1 + a5) * w1 +
                  (a2 + a6) * w2 + (a3 + a7) * w3) * inv_l + bv
            for k in (1, 2, 4, 8):
                shuf = lax.gather(zv, (lane ^ k)[:, None], dnums,
                                  slice_sizes=(1,),
                                  mode=lax.GatherScatterMode.PROMISE_IN_BOUNDS)
                zv = zv + shuf
            zgroup = jnp.where(lane == p, zv, zgroup)

        out_v[pl.ds(g * 16, 16)] = 1.0 / (1.0 + jnp.exp(-zgroup))
        return carry

    lax.fori_loop(0, BPW // 16, group_body, 0)
    pltpu.sync_copy(out_v, out_hbm.at[pl.ds(base, BPW)])


def kernel(x, embed_table, fc_w, fc_b):
    x3 = x.astype(jnp.int32).reshape(B, 2, HALF)
    w = fc_w.reshape(D).astype(jnp.float32)
    bpad = jnp.pad(fc_b.astype(jnp.float32), (0, 15))
    mesh = plsc.VectorSubcoreMesh(core_axis_name="c", subcore_axis_name="s")
    run = pl.kernel(
        _disc_body,
        out_type=jax.ShapeDtypeStruct((B,), jnp.float32),
        mesh=mesh,
        compiler_params=pltpu.CompilerParams(use_tc_tiling_on_sc=False),
        scratch_types=[
            pltpu.VMEM((BPW, 2, HALF), jnp.int32),
            pltpu.VMEM((L, D), jnp.float32),
            pltpu.VMEM((L, D), jnp.float32),
            pltpu.VMEM((L, D), jnp.float32),
            pltpu.VMEM((L, D), jnp.float32),
            pltpu.VMEM((D,), jnp.float32),
            pltpu.VMEM((16,), jnp.float32),
            pltpu.VMEM((BPW,), jnp.float32),
            pltpu.SemaphoreType.DMA,
            pltpu.SemaphoreType.DMA,
            pltpu.SemaphoreType.DMA,
            pltpu.SemaphoreType.DMA,
        ],
    )
    out = run(x3, embed_table, w, bpad)
    return out.reshape(B, 1)


# speedup vs baseline: 3.0726x; 2.4741x over previous
import jax
import jax.numpy as jnp
from jax import lax
from jax.experimental import pallas as pl
from jax.experimental.pallas import tpu as pltpu
from jax.experimental.pallas import tpu_sc as plsc

B = 4096
L = 200
D = 64
VOCAB = 1000000
NC = 2
NS = 16
NW = NC * NS
BPW = B // NW
LP = 208
CPG = 16 * LP // 128
GPW = BPW // 16
NBUF = 4

VB = 32768


def _scores_body(tt_ref, w_ref, s_ref):
    s_ref[...] = jnp.sum(tt_ref[...] * w_ref[...], axis=0)


def _pool_body(idx_hbm, scores_hbm, bias_hbm, out_hbm,
               idx_v, buf0, buf1, buf2, buf3, bias_v, out_v,
               sem0, sem1, sem2, sem3):
    c = lax.axis_index("c")
    s = lax.axis_index("s")
    wid = s * NC + c

    bufs = ((buf0, sem0), (buf1, sem1), (buf2, sem2), (buf3, sem3))

    pltpu.sync_copy(idx_hbm.at[wid], idx_v)
    pltpu.sync_copy(bias_hbm, bias_v)

    bv = bias_v[...]
    lane = lax.iota(jnp.int32, 16)
    zero = jnp.zeros((16,), jnp.float32)
    tail_mask = lane < 8
    inv_l = jnp.float32(1.0 / L)
    dnums = lax.GatherDimensionNumbers(
        offset_dims=(), collapsed_slice_dims=(0,), start_index_map=(0,))

    def fire(g, buf, sem):
        for k in range(CPG):
            pltpu.async_copy(scores_hbm.at[idx_v.at[g * CPG + k]],
                             buf.at[pl.ds(k * 128, 128)], sem)

    def drain(buf, sem):
        pltpu.make_async_copy(scores_hbm.at[pl.ds(0, 16 * LP)], buf,
                              sem).wait()

    for p in range(NBUF - 1):
        fire(p, *bufs[p])

    def group4_body(g2, carry):
        for p in range(NBUF):
            g = g2 * NBUF + p
            nxt = g + NBUF - 1
            buf, sem = bufs[p]

            @pl.when(nxt < GPW)
            def _():
                fire(nxt, *bufs[(p + NBUF - 1) % NBUF])

            drain(buf, sem)

            zgroup = zero
            for j in range(16):
                acc = jnp.where(tail_mask, buf[pl.ds(j * LP + 192, 16)], 0.0)
                for i in range(12):
                    acc = acc + buf[pl.ds(j * LP + i * 16, 16)]
                zv = acc * inv_l + bv
                for k in (1, 2, 4, 8):
                    shuf = lax.gather(
                        zv, (lane ^ k)[:, None], dnums, slice_sizes=(1,),
                        mode=lax.GatherScatterMode.PROMISE_IN_BOUNDS)
                    zv = zv + shuf
                zgroup = jnp.where(lane == j, zv, zgroup)

            out_v[pl.ds(g * 16, 16)] = 1.0 / (1.0 + jnp.exp(-zgroup))
        return carry

    lax.fori_loop(0, GPW // NBUF, group4_body, 0)
    pltpu.sync_copy(out_v, out_hbm.at[pl.ds(wid * BPW, BPW)])


def kernel(x, embed_table, fc_w, fc_b):
    wcol = fc_w.reshape(D, 1).astype(jnp.float32)
    scores = pl.pallas_call(
        _scores_body,
        grid=(pl.cdiv(VOCAB, VB),),
        in_specs=[
            pl.BlockSpec((D, VB), lambda i: (0, i)),
            pl.BlockSpec((D, 1), lambda i: (0, 0)),
        ],
        out_specs=pl.BlockSpec((VB,), lambda i: (i,)),
        out_shape=jax.ShapeDtypeStruct((VOCAB,), jnp.float32),
    )(embed_table.T, wcol)

    xpad = jnp.pad(x.astype(jnp.int32), ((0, 0), (0, LP - L)))
    idx3 = xpad.reshape(NW, GPW * CPG, 128)
    bpad = jnp.pad(fc_b.astype(jnp.float32), (0, 15))

    mesh = plsc.VectorSubcoreMesh(core_axis_name="c", subcore_axis_name="s")
    run = pl.kernel(
        _pool_body,
        out_type=jax.ShapeDtypeStruct((B,), jnp.float32),
        mesh=mesh,
        compiler_params=pltpu.CompilerParams(use_tc_tiling_on_sc=False),
        scratch_types=[
            pltpu.VMEM((GPW * CPG, 128), jnp.int32),
            pltpu.VMEM((16 * LP,), jnp.float32),
            pltpu.VMEM((16 * LP,), jnp.float32),
            pltpu.VMEM((16 * LP,), jnp.float32),
            pltpu.VMEM((16 * LP,), jnp.float32),
            pltpu.VMEM((16,), jnp.float32),
            pltpu.VMEM((BPW,), jnp.float32),
            pltpu.SemaphoreType.DMA,
            pltpu.SemaphoreType.DMA,
            pltpu.SemaphoreType.DMA,
            pltpu.SemaphoreType.DMA,
        ],
    )
    out = run(idx3, scores, bpad)
    return out.reshape(B, 1)
